# baseline (speedup 1.0000x reference)
import functools

import jax
import jax.numpy as jnp
from jax import lax
from jax.experimental import pallas as pl
from jax.experimental.pallas import tpu as pltpu
from jax.experimental.pallas import tpu_sc as plsc

D = 64
NW = 32
G = 128


def _make_sc_gather(n_rows: int):
    per_w = n_rows // NW
    ng = per_w // G
    mesh = plsc.VectorSubcoreMesh(core_axis_name="c", subcore_axis_name="s")

    @functools.partial(
        pl.kernel,
        out_type=jax.ShapeDtypeStruct((n_rows, D), jnp.float32),
        mesh=mesh,
        compiler_params=pltpu.CompilerParams(use_tc_tiling_on_sc=False),
        scratch_types=[
            pltpu.VMEM((ng, G), jnp.int32),
            pltpu.VMEM((G, D), jnp.float32),
            pltpu.VMEM((G, D), jnp.float32),
            pltpu.SemaphoreType.DMA,
            pltpu.SemaphoreType.DMA,
        ],
    )
    def emb(tok_hbm, table_hbm, out_hbm, idx_v, rows0, rows1, sem0, sem1):
        wid = lax.axis_index("s") * 2 + lax.axis_index("c")
        base = wid * per_w
        pltpu.sync_copy(tok_hbm.at[wid], idx_v)
        pltpu.async_copy(table_hbm.at[idx_v.at[0]], rows0, sem0)

        @pl.loop(0, ng, step=2)
        def _(g):
            pltpu.make_async_copy(table_hbm.at[idx_v.at[g]], rows0, sem0).wait()
            pltpu.async_copy(table_hbm.at[idx_v.at[g + 1]], rows1, sem1)
            pltpu.sync_copy(rows0, out_hbm.at[pl.ds(base + g * G, G)])
            pltpu.make_async_copy(
                table_hbm.at[idx_v.at[g + 1]], rows1, sem1).wait()

            @pl.when(g + 2 < ng)
            def _():
                pltpu.async_copy(table_hbm.at[idx_v.at[g + 2]], rows0, sem0)

            pltpu.sync_copy(rows1, out_hbm.at[pl.ds(base + (g + 1) * G, G)])

    return emb


def kernel(tokens, weights):
    batch, hist = tokens.shape
    n_rows = batch * hist
    tok = tokens.astype(jnp.int32).reshape(NW, n_rows // (NW * G), G)
    out = _make_sc_gather(n_rows)(tok, weights)
    return out.reshape(batch, hist, D)


# speedup vs baseline: 1.0148x; 1.0148x over previous
import functools

import jax
import jax.numpy as jnp
from jax import lax
from jax.experimental import pallas as pl
from jax.experimental.pallas import tpu as pltpu
from jax.experimental.pallas import tpu_sc as plsc

D = 64
NW = 32
G = 128


def _make_sc_gather(n_rows: int):
    per_w = n_rows // NW
    ng = per_w // G
    mesh = plsc.VectorSubcoreMesh(core_axis_name="c", subcore_axis_name="s")

    @functools.partial(
        pl.kernel,
        out_type=jax.ShapeDtypeStruct((n_rows, D), jnp.float32),
        mesh=mesh,
        compiler_params=pltpu.CompilerParams(use_tc_tiling_on_sc=False),
        scratch_types=[
            pltpu.VMEM((ng, G), jnp.int32),
            pltpu.VMEM((G, D), jnp.float32),
            pltpu.VMEM((G, D), jnp.float32),
            pltpu.SemaphoreType.DMA,
            pltpu.SemaphoreType.DMA,
        ],
    )
    def emb(tok_hbm, table_hbm, out_hbm, idx_v, rows0, rows1, sem0, sem1):
        wid = lax.axis_index("s") * 2 + lax.axis_index("c")
        pltpu.sync_copy(tok_hbm.at[:, wid], idx_v)
        pltpu.async_copy(table_hbm.at[idx_v.at[0]], rows0, sem0)

        @pl.loop(0, ng, step=2)
        def _(g):
            pltpu.make_async_copy(table_hbm.at[idx_v.at[g]], rows0, sem0).wait()
            pltpu.async_copy(table_hbm.at[idx_v.at[g + 1]], rows1, sem1)
            pltpu.sync_copy(rows0, out_hbm.at[pl.ds(g * (NW * G) + wid * G, G)])
            pltpu.make_async_copy(
                table_hbm.at[idx_v.at[g + 1]], rows1, sem1).wait()

            @pl.when(g + 2 < ng)
            def _():
                pltpu.async_copy(table_hbm.at[idx_v.at[g + 2]], rows0, sem0)

            pltpu.sync_copy(
                rows1, out_hbm.at[pl.ds((g + 1) * (NW * G) + wid * G, G)])

    return emb


def kernel(tokens, weights):
    batch, hist = tokens.shape
    n_rows = batch * hist
    tok = tokens.T.astype(jnp.int32).reshape(hist, NW, G)
    out = _make_sc_gather(n_rows)(tok, weights)
    return out.reshape(hist, batch, D).transpose(1, 0, 2)
